# Initial kernel scaffold; baseline (speedup 1.0000x reference)
import jax
import jax.numpy as jnp
from jax.experimental import pallas as pl


def kernel(x, main_flow, combinator):
    raise NotImplementedError("write your pallas kernel here")


# speedup vs baseline: 55.0144x; 55.0144x over previous
import functools

import jax
import jax.numpy as jnp
from jax import lax
from jax.experimental import pallas as pl
from jax.experimental.pallas import tpu as pltpu
from jax.experimental.pallas import tpu_sc as plsc

_R, _C, _KF = 450, 480, 10
_NFLOW = 8
_NFEAT = 8
_NOUT = 32
_NB = 2
_P = _R * _C
_NCH = _NB * _NFEAT
_ROWS_PER_BLK = 10
_PT = 600
_GK = 3000


def _idx_weight_kernel(mf_ref, idx_ref, w_ref):
    pid = pl.program_id(0)
    kk = lax.broadcasted_iota(jnp.int32, (_KF * _KF, _C), 0)
    k1 = (kk // _KF).astype(jnp.float32)
    k2 = (kk % _KF).astype(jnp.float32)
    cc = lax.broadcasted_iota(jnp.int32, (_KF * _KF, _C), 1).astype(jnp.float32)
    jjn = cc / _C
    cvec = lax.broadcasted_iota(jnp.int32, (_NFLOW, _C), 1).astype(jnp.float32)
    for j in range(_ROWS_PER_BLK):
        row = pid * _ROWS_PER_BLK + j
        rowf = row.astype(jnp.float32)
        iin = rowf / _R
        phase = 2.0 * jnp.pi * (k1 * iin + k2 * jjn)
        basis = jnp.cos(phase)
        fr = lax.dot_general(mf_ref[0], basis, (((1,), (0,)), ((), ())),
                             preferred_element_type=jnp.float32)
        fc = lax.dot_general(mf_ref[1], basis, (((1,), (0,)), ((), ())),
                             preferred_element_type=jnp.float32)
        rm = rowf + (-fr)
        cm = cvec + (-fc)
        b0r = jnp.round(rm)
        b0c = jnp.round(cm)
        s = rm - jnp.floor(rm)
        t = cm - jnp.floor(cm)
        x0 = jnp.clip(b0r, 0, _R - 1).astype(jnp.int32)
        x1 = jnp.clip(b0r + 1.0, 0, _R - 1).astype(jnp.int32)
        y0 = jnp.clip(b0c, 0, _C - 1).astype(jnp.int32)
        y1 = jnp.clip(b0c + 1.0, 0, _C - 1).astype(jnp.int32)
        idx = jnp.concatenate(
            [x0 * _C + y0, x1 * _C + y0, x0 * _C + y1, x1 * _C + y1], axis=1)
        w = jnp.concatenate(
            [s * t, s * (1.0 - t), (1.0 - s) * t, (1.0 - s) * (1.0 - t)],
            axis=1)
        idx_ref[j] = idx
        w_ref[j] = w


def _compute_idx_weights(mf_t):
    grid = _R // _ROWS_PER_BLK
    return pl.pallas_call(
        _idx_weight_kernel,
        grid=(grid,),
        in_specs=[pl.BlockSpec((2, _NFLOW, _KF * _KF), lambda i: (0, 0, 0))],
        out_specs=[
            pl.BlockSpec((_ROWS_PER_BLK, _NFLOW, 4 * _C), lambda i: (i, 0, 0)),
            pl.BlockSpec((_ROWS_PER_BLK, _NFLOW, 4 * _C), lambda i: (i, 0, 0)),
        ],
        out_shape=[
            jax.ShapeDtypeStruct((_R, _NFLOW, 4 * _C), jnp.int32),
            jax.ShapeDtypeStruct((_R, _NFLOW, 4 * _C), jnp.float32),
        ],
    )(mf_t)


def _sc_gather(table, idx_flat):
    ntot = idx_flat.shape[0]
    info = plsc.get_sparse_core_info()
    nw = info.num_cores * info.num_subcores
    per_w = ntot // nw
    iters = per_w // _GK
    mesh = plsc.VectorSubcoreMesh(core_axis_name="c", subcore_axis_name="s")

    @functools.partial(
        pl.kernel,
        mesh=mesh,
        compiler_params=pltpu.CompilerParams(use_tc_tiling_on_sc=False),
        out_type=jax.ShapeDtypeStruct((ntot, _NCH), jnp.float32),
        scratch_types=[
            pltpu.VMEM((_GK,), jnp.int32),
            pltpu.VMEM((_GK, _NCH), jnp.float32),
            pltpu.SemaphoreType.DMA,
        ],
    )
    def gather_k(table_hbm, idx_hbm, out_hbm, idx_v, rows_v, sem):
        wid = lax.axis_index("s") * info.num_cores + lax.axis_index("c")
        base = wid * per_w

        def body(i, carry):
            off = pl.multiple_of(base + i * _GK, 8)
            pltpu.sync_copy(idx_hbm.at[pl.ds(off, _GK)], idx_v)
            pltpu.async_copy(table_hbm.at[idx_v], rows_v, sem).wait()
            pltpu.sync_copy(rows_v, out_hbm.at[pl.ds(off, _GK)])
            return carry

        lax.fori_loop(0, iters, body, 0)

    return gather_k(table, idx_flat)


def _combine_kernel(g_ref, w_ref, comb_ref, out_ref):
    comb = comb_ref[...]
    g = g_ref[0]
    wv = w_ref[0]
    accs = []
    for b in range(_NB):
        acc = jnp.zeros((_PT, _NOUT), jnp.float32)
        for n in range(_NFLOW):
            wn = jnp.zeros((_PT, _NFEAT), jnp.float32)
            for cr in range(4):
                i = n * 4 + cr
                wn = wn + wv[i][:, None] * g[i][:, b * _NFEAT:(b + 1) * _NFEAT]
            acc = acc + lax.dot_general(
                wn, comb[:, n * _NFEAT:(n + 1) * _NFEAT],
                (((1,), (1,)), ((), ())),
                preferred_element_type=jnp.float32)
        accs.append(acc)
    out_ref[...] = jnp.concatenate(accs, axis=1)


def _combine(g, w, comb):
    grid = _P // _PT
    return pl.pallas_call(
        _combine_kernel,
        grid=(grid,),
        in_specs=[
            pl.BlockSpec((1, _NFLOW * 4, _PT, _NCH), lambda i: (i, 0, 0, 0)),
            pl.BlockSpec((1, _NFLOW * 4, _PT), lambda i: (i, 0, 0)),
            pl.BlockSpec((_NOUT, _NFLOW * _NFEAT), lambda i: (0, 0)),
        ],
        out_specs=pl.BlockSpec((_PT, _NB * _NOUT), lambda i: (i, 0)),
        out_shape=jax.ShapeDtypeStruct((_P, _NB * _NOUT), jnp.float32),
    )(g, w, comb)


def kernel(x, main_flow, combinator):
    xt = jnp.transpose(x, (2, 3, 0, 1)).reshape(_P, _NCH)
    mf_t = jnp.transpose(main_flow, (2, 0, 1))
    idx_a, w_a = _compute_idx_weights(mf_t)
    nt = _P // _PT

    def _reorder(a):
        a = a.reshape(_R, _NFLOW, 4, _C).transpose(1, 2, 0, 3)
        return a.reshape(_NFLOW * 4, nt, _PT).transpose(1, 0, 2)

    idx3 = _reorder(idx_a)
    w3 = _reorder(w_a)
    g = _sc_gather(xt, idx3.reshape(-1))
    out = _combine(g.reshape(nt, _NFLOW * 4, _PT, _NCH), w3, combinator)
    out = out.reshape(_R, _C, _NB, _NOUT)
    return jnp.transpose(out, (2, 3, 0, 1))


# speedup vs baseline: 56.4824x; 1.0267x over previous
import functools

import jax
import jax.numpy as jnp
from jax import lax
from jax.experimental import pallas as pl
from jax.experimental.pallas import tpu as pltpu
from jax.experimental.pallas import tpu_sc as plsc

_R, _C, _KF = 450, 480, 10
_NFLOW = 8
_NFEAT = 8
_NOUT = 32
_NB = 2
_P = _R * _C
_NCH = _NB * _NFEAT
_ROWS_PER_BLK = 10
_PT = 600
_GK = 3000


def _idx_weight_kernel(mf_ref, idx_ref, w_ref):
    pid = pl.program_id(0)
    kk = lax.broadcasted_iota(jnp.int32, (_KF * _KF, _C), 0)
    k1 = (kk // _KF).astype(jnp.float32)
    k2 = (kk % _KF).astype(jnp.float32)
    cc = lax.broadcasted_iota(jnp.int32, (_KF * _KF, _C), 1).astype(jnp.float32)
    jjn = cc / _C
    cvec = lax.broadcasted_iota(jnp.int32, (_NFLOW, _C), 1).astype(jnp.float32)
    for j in range(_ROWS_PER_BLK):
        row = pid * _ROWS_PER_BLK + j
        rowf = row.astype(jnp.float32)
        iin = rowf / _R
        phase = 2.0 * jnp.pi * (k1 * iin + k2 * jjn)
        basis = jnp.cos(phase)
        fr = lax.dot_general(mf_ref[0], basis, (((1,), (0,)), ((), ())),
                             preferred_element_type=jnp.float32)
        fc = lax.dot_general(mf_ref[1], basis, (((1,), (0,)), ((), ())),
                             preferred_element_type=jnp.float32)
        rm = rowf + (-fr)
        cm = cvec + (-fc)
        b0r = jnp.round(rm)
        b0c = jnp.round(cm)
        s = rm - jnp.floor(rm)
        t = cm - jnp.floor(cm)
        x0 = jnp.clip(b0r, 0, _R - 1).astype(jnp.int32)
        x1 = jnp.clip(b0r + 1.0, 0, _R - 1).astype(jnp.int32)
        y0 = jnp.clip(b0c, 0, _C - 1).astype(jnp.int32)
        y1 = jnp.clip(b0c + 1.0, 0, _C - 1).astype(jnp.int32)
        idx = jnp.concatenate(
            [x0 * _C + y0, x1 * _C + y0, x0 * _C + y1, x1 * _C + y1], axis=1)
        w = jnp.concatenate(
            [s * t, s * (1.0 - t), (1.0 - s) * t, (1.0 - s) * (1.0 - t)],
            axis=1)
        idx_ref[j] = idx
        w_ref[j] = w


def _compute_idx_weights(mf_t):
    grid = _R // _ROWS_PER_BLK
    return pl.pallas_call(
        _idx_weight_kernel,
        grid=(grid,),
        in_specs=[pl.BlockSpec((2, _NFLOW, _KF * _KF), lambda i: (0, 0, 0))],
        out_specs=[
            pl.BlockSpec((_ROWS_PER_BLK, _NFLOW, 4 * _C), lambda i: (i, 0, 0)),
            pl.BlockSpec((_ROWS_PER_BLK, _NFLOW, 4 * _C), lambda i: (i, 0, 0)),
        ],
        out_shape=[
            jax.ShapeDtypeStruct((_R, _NFLOW, 4 * _C), jnp.int32),
            jax.ShapeDtypeStruct((_R, _NFLOW, 4 * _C), jnp.float32),
        ],
    )(mf_t)


def _sc_gather(table, idx_flat):
    ntot = idx_flat.shape[0]
    info = plsc.get_sparse_core_info()
    nw = info.num_cores * info.num_subcores
    per_w = ntot // nw
    iters = per_w // _GK
    mesh = plsc.VectorSubcoreMesh(core_axis_name="c", subcore_axis_name="s")

    @functools.partial(
        pl.kernel,
        mesh=mesh,
        compiler_params=pltpu.CompilerParams(use_tc_tiling_on_sc=False),
        out_type=jax.ShapeDtypeStruct((ntot, _NCH), jnp.float32),
        scratch_types=[
            pltpu.VMEM((_GK,), jnp.int32),
            pltpu.VMEM((_GK, _NCH), jnp.float32),
            pltpu.SemaphoreType.DMA,
        ],
    )
    def gather_k(table_hbm, idx_hbm, out_hbm, idx_v, rows_v, sem):
        wid = lax.axis_index("s") * info.num_cores + lax.axis_index("c")
        base = wid * per_w

        def body(i, carry):
            off = pl.multiple_of(base + i * _GK, 8)
            pltpu.sync_copy(idx_hbm.at[pl.ds(off, _GK)], idx_v)
            pltpu.async_copy(table_hbm.at[idx_v], rows_v, sem).wait()
            pltpu.sync_copy(rows_v, out_hbm.at[pl.ds(off, _GK)])
            return carry

        lax.fori_loop(0, iters, body, 0)

    return gather_k(table, idx_flat)


_KDIM = _NFLOW * 4 * _NCH


def _combine_kernel(g_ref, w_ref, m_ref, out_ref):
    a = g_ref[0]
    wt = w_ref[0]
    ri = lax.broadcasted_iota(jnp.int32, (_NFLOW * 4, _KDIM), 0)
    ci = lax.broadcasted_iota(jnp.int32, (_NFLOW * 4, _KDIM), 1)
    rep = (ci // _NCH == ri).astype(jnp.float32)
    w_rep = lax.dot_general(wt, rep, (((1,), (0,)), ((), ())),
                            preferred_element_type=jnp.float32)
    out_ref[...] = lax.dot_general(a * w_rep, m_ref[...],
                                   (((1,), (0,)), ((), ())),
                                   preferred_element_type=jnp.float32)


def _combine(g, w, m):
    grid = _P // _PT
    return pl.pallas_call(
        _combine_kernel,
        grid=(grid,),
        in_specs=[
            pl.BlockSpec((1, _PT, _KDIM), lambda i: (i, 0, 0)),
            pl.BlockSpec((1, _PT, _NFLOW * 4), lambda i: (i, 0, 0)),
            pl.BlockSpec((_KDIM, _NB * _NOUT), lambda i: (0, 0)),
        ],
        out_specs=pl.BlockSpec((_PT, _NB * _NOUT), lambda i: (i, 0)),
        out_shape=jax.ShapeDtypeStruct((_P, _NB * _NOUT), jnp.float32),
    )(g, w, m)


def kernel(x, main_flow, combinator):
    xt = jnp.transpose(x, (2, 3, 0, 1)).reshape(_P, _NCH)
    mf_t = jnp.transpose(main_flow, (2, 0, 1))
    idx_a, w_a = _compute_idx_weights(mf_t)
    nt = _P // _PT

    def _reorder(a):
        a = a.reshape(_R, _NFLOW, 4, _C).transpose(0, 3, 1, 2)
        return a.reshape(nt, _PT, _NFLOW * 4)

    idx3 = _reorder(idx_a)
    w3 = _reorder(w_a)
    combr = combinator.reshape(_NOUT, _NFLOW, _NFEAT)
    m = (combr.transpose(1, 2, 0)[:, None, None, :, None, :]
         * jnp.eye(_NB, dtype=jnp.float32)[None, None, :, None, :, None])
    m = jnp.broadcast_to(m, (_NFLOW, 4, _NB, _NFEAT, _NB, _NOUT))
    m = m.reshape(_KDIM, _NB * _NOUT)
    g = _sc_gather(xt, idx3.reshape(-1))
    out = _combine(g.reshape(nt, _PT, _KDIM), w3, m)
    out = out.reshape(_R, _C, _NB, _NOUT)
    return jnp.transpose(out, (2, 3, 0, 1))


# speedup vs baseline: 57.2329x; 1.0133x over previous
import functools

import jax
import jax.numpy as jnp
from jax import lax
from jax.experimental import pallas as pl
from jax.experimental.pallas import tpu as pltpu
from jax.experimental.pallas import tpu_sc as plsc

_R, _C, _KF = 450, 480, 10
_NFLOW = 8
_NFEAT = 8
_NOUT = 32
_NB = 2
_P = _R * _C
_NCH = _NB * _NFEAT
_ROWS_PER_BLK = 10
_PT = 600
_GK = 3000


def _idx_weight_kernel(mf_ref, idx_ref, w_ref):
    pid = pl.program_id(0)
    kk = lax.broadcasted_iota(jnp.int32, (_KF * _KF, _C), 0)
    k1 = (kk // _KF).astype(jnp.float32)
    k2 = (kk % _KF).astype(jnp.float32)
    cc = lax.broadcasted_iota(jnp.int32, (_KF * _KF, _C), 1).astype(jnp.float32)
    jjn = cc / _C
    cvec = lax.broadcasted_iota(jnp.int32, (_NFLOW, _C), 1).astype(jnp.float32)
    for j in range(_ROWS_PER_BLK):
        row = pid * _ROWS_PER_BLK + j
        rowf = row.astype(jnp.float32)
        iin = rowf / _R
        phase = 2.0 * jnp.pi * (k1 * iin + k2 * jjn)
        basis = jnp.cos(phase)
        fr = lax.dot_general(mf_ref[0], basis, (((1,), (0,)), ((), ())),
                             preferred_element_type=jnp.float32)
        fc = lax.dot_general(mf_ref[1], basis, (((1,), (0,)), ((), ())),
                             preferred_element_type=jnp.float32)
        rm = rowf + (-fr)
        cm = cvec + (-fc)
        b0r = jnp.round(rm)
        b0c = jnp.round(cm)
        s = rm - jnp.floor(rm)
        t = cm - jnp.floor(cm)
        x0 = jnp.clip(b0r, 0, _R - 1).astype(jnp.int32)
        x1 = jnp.clip(b0r + 1.0, 0, _R - 1).astype(jnp.int32)
        y0 = jnp.clip(b0c, 0, _C - 1).astype(jnp.int32)
        y1 = jnp.clip(b0c + 1.0, 0, _C - 1).astype(jnp.int32)
        idx = jnp.concatenate(
            [x0 * _C + y0, x1 * _C + y0, x0 * _C + y1, x1 * _C + y1], axis=1)
        w = jnp.concatenate(
            [s * t, s * (1.0 - t), (1.0 - s) * t, (1.0 - s) * (1.0 - t)],
            axis=1)
        idx_ref[j] = idx
        w_ref[j] = w


def _compute_idx_weights(mf_t):
    grid = _R // _ROWS_PER_BLK
    return pl.pallas_call(
        _idx_weight_kernel,
        grid=(grid,),
        in_specs=[pl.BlockSpec((2, _NFLOW, _KF * _KF), lambda i: (0, 0, 0))],
        out_specs=[
            pl.BlockSpec((_ROWS_PER_BLK, _NFLOW, 4 * _C), lambda i: (i, 0, 0)),
            pl.BlockSpec((_ROWS_PER_BLK, _NFLOW, 4 * _C), lambda i: (i, 0, 0)),
        ],
        out_shape=[
            jax.ShapeDtypeStruct((_R, _NFLOW, 4 * _C), jnp.int32),
            jax.ShapeDtypeStruct((_R, _NFLOW, 4 * _C), jnp.float32),
        ],
    )(mf_t)


def _sc_gather(table, idx_flat):
    ntot = idx_flat.shape[0]
    info = plsc.get_sparse_core_info()
    nw = info.num_cores * info.num_subcores
    per_w = ntot // nw
    iters = per_w // _GK
    mesh = plsc.VectorSubcoreMesh(core_axis_name="c", subcore_axis_name="s")

    @functools.partial(
        pl.kernel,
        mesh=mesh,
        compiler_params=pltpu.CompilerParams(use_tc_tiling_on_sc=False),
        out_type=jax.ShapeDtypeStruct((ntot, _NCH), jnp.float32),
        scratch_types=[
            pltpu.VMEM((_GK,), jnp.int32),
            pltpu.VMEM((_GK, _NCH), jnp.float32),
            pltpu.SemaphoreType.DMA,
        ],
    )
    def gather_k(table_hbm, idx_hbm, out_hbm, idx_v, rows_v, sem):
        wid = lax.axis_index("s") * info.num_cores + lax.axis_index("c")
        base = wid * per_w

        def body(i, carry):
            off = pl.multiple_of(base + i * _GK, 8)
            pltpu.sync_copy(idx_hbm.at[pl.ds(off, _GK)], idx_v)
            pltpu.async_copy(table_hbm.at[idx_v], rows_v, sem).wait()
            pltpu.sync_copy(rows_v, out_hbm.at[pl.ds(off, _GK)])
            return carry

        lax.fori_loop(0, iters, body, 0)

    return gather_k(table, idx_flat)


_KDIM = _NFLOW * 4 * _NCH


def _combine_kernel(g_ref, w_ref, m_ref, out_ref):
    a = g_ref[0]
    wt = w_ref[0]
    ri = lax.broadcasted_iota(jnp.int32, (_NFLOW * 4, _KDIM), 0)
    ci = lax.broadcasted_iota(jnp.int32, (_NFLOW * 4, _KDIM), 1)
    rep = (ci // _NCH == ri).astype(jnp.float32)
    w_rep = lax.dot_general(wt, rep, (((1,), (0,)), ((), ())),
                            preferred_element_type=jnp.float32)
    out_ref[0] = lax.dot_general(m_ref[...], a * w_rep,
                                 (((0,), (1,)), ((), ())),
                                 preferred_element_type=jnp.float32)


def _combine(g, w, m):
    grid = _P // _PT
    return pl.pallas_call(
        _combine_kernel,
        grid=(grid,),
        in_specs=[
            pl.BlockSpec((1, _PT, _KDIM), lambda i: (i, 0, 0)),
            pl.BlockSpec((1, _PT, _NFLOW * 4), lambda i: (i, 0, 0)),
            pl.BlockSpec((_KDIM, _NB * _NOUT), lambda i: (0, 0)),
        ],
        out_specs=pl.BlockSpec((1, _NB * _NOUT, _PT), lambda i: (i, 0, 0)),
        out_shape=jax.ShapeDtypeStruct((_P // _PT, _NB * _NOUT, _PT),
                                       jnp.float32),
    )(g, w, m)


def kernel(x, main_flow, combinator):
    xt = jnp.transpose(x, (2, 3, 0, 1)).reshape(_P, _NCH)
    mf_t = jnp.transpose(main_flow, (2, 0, 1))
    idx_a, w_a = _compute_idx_weights(mf_t)
    nt = _P // _PT

    def _reorder(a):
        a = a.reshape(_R, _NFLOW, 4, _C).transpose(0, 3, 1, 2)
        return a.reshape(nt, _PT, _NFLOW * 4)

    idx3 = _reorder(idx_a)
    w3 = _reorder(w_a)
    combr = combinator.reshape(_NOUT, _NFLOW, _NFEAT)
    m = (combr.transpose(1, 2, 0)[:, None, None, :, None, :]
         * jnp.eye(_NB, dtype=jnp.float32)[None, None, :, None, :, None])
    m = jnp.broadcast_to(m, (_NFLOW, 4, _NB, _NFEAT, _NB, _NOUT))
    m = m.reshape(_KDIM, _NB * _NOUT)
    g = _sc_gather(xt, idx3.reshape(-1))
    out = _combine(g.reshape(nt, _PT, _KDIM), w3, m)
    out = jnp.transpose(out, (1, 0, 2))
    return out.reshape(_NB, _NOUT, _R, _C)
